# baseline (speedup 1.0000x reference)
import functools

import jax
import jax.numpy as jnp
from jax.experimental import pallas as pl
from jax.experimental.pallas import tpu as pltpu

_VALUE_ITERS = 32
_SIZE_ITERS = 18
_INDEX_ITERS = 18


def _conformal_body(x_ref, p_ref, o_ref):
    invT = p_ref[0, 0]
    qhat = p_ref[0, 1]
    lam = p_ref[0, 2]
    kreg = p_ref[0, 3]

    x = x_ref[...] * invT
    m = jnp.max(x, axis=1, keepdims=True)
    e = jnp.exp(x - m)
    z = jnp.sum(e, axis=1, keepdims=True)
    zq = qhat * z
    zlam = lam * z

    lo = jnp.zeros_like(z)
    hi = jnp.full_like(z, 1.5)

    def vstep(_, carry):
        lo, hi = carry
        mid = 0.5 * (lo + hi)
        pred = e >= mid
        n = jnp.sum(jnp.where(pred, 1.0, 0.0), axis=1, keepdims=True)
        s = jnp.sum(jnp.where(pred, e, 0.0), axis=1, keepdims=True)
        g = s + zlam * jnp.maximum(n - kreg, 0.0)
        big = g > zq
        lo = jnp.where(big, mid, lo)
        hi = jnp.where(big, hi, mid)
        return lo, hi

    lo, hi = jax.lax.fori_loop(0, _VALUE_ITERS, vstep, (lo, hi))

    inc = e >= lo
    v = jnp.min(jnp.where(inc, e, 2.0), axis=1, keepdims=True)
    strict = e > v
    n_strict = jnp.sum(jnp.where(strict, 1.0, 0.0), axis=1, keepdims=True)
    s_strict = jnp.sum(jnp.where(strict, e, 0.0), axis=1, keepdims=True)
    n_total = jnp.sum(jnp.where(e >= v, 1.0, 0.0), axis=1, keepdims=True)

    def nstep(_, carry):
        lo_n, hi_n = carry
        mid = jnp.floor(0.5 * (lo_n + hi_n))
        g = s_strict + (mid - n_strict) * v + zlam * jnp.maximum(mid - kreg, 0.0)
        big = (g > zq) & (mid > lo_n)
        hi_n = jnp.where(big, mid, hi_n)
        lo_n = jnp.where(big | (mid <= lo_n), lo_n, mid)
        return lo_n, hi_n

    _, n_star = jax.lax.fori_loop(0, _SIZE_ITERS, nstep, (n_strict, n_total))
    needed = n_star - n_strict

    iota = jax.lax.broadcasted_iota(jnp.int32, e.shape, 1).astype(jnp.float32)
    tie = e == v
    full_cut = jnp.full_like(z, float(e.shape[1]))

    def _bisect_cutoff():
        def istep(_, carry):
            lo_i, hi_i = carry
            mid = jnp.floor(0.5 * (lo_i + hi_i))
            cnt = jnp.sum(jnp.where(tie & (iota < mid), 1.0, 0.0), axis=1,
                          keepdims=True)
            ok = (cnt >= needed) & (mid > lo_i)
            hi_i = jnp.where(ok, mid, hi_i)
            lo_i = jnp.where(ok | (mid <= lo_i), lo_i, mid)
            return lo_i, hi_i

        return jax.lax.fori_loop(0, _INDEX_ITERS, istep,
                                 (jnp.zeros_like(z), full_cut))[1]

    has_partial = jnp.any(needed < n_total - n_strict)
    cutoff = jax.lax.cond(has_partial, _bisect_cutoff, lambda: full_cut)

    mask = (strict | (tie & (iota < cutoff))).astype(jnp.float32)
    mask = jnp.where(qhat == 1.0, 1.0, mask)
    o_ref[...] = mask


@functools.partial(jax.jit, static_argnames=())
def kernel(logits, temperature, penalties, qhat):
    B, V = logits.shape
    Vp = ((V + 127) // 128) * 128
    BR = 8

    lam = penalties[0, -1]
    kreg = jnp.sum(penalties[0] == 0).astype(jnp.float32)
    invT = 1.0 / temperature[0]
    params = jnp.stack(
        [invT, qhat.astype(jnp.float32), lam, kreg]
    ).reshape(1, 4)

    xp = jnp.pad(logits, ((0, 0), (0, Vp - V)), constant_values=-jnp.inf)

    mask = pl.pallas_call(
        _conformal_body,
        grid=(B // BR,),
        in_specs=[
            pl.BlockSpec((BR, Vp), lambda i: (i, 0)),
            pl.BlockSpec((1, 4), lambda i: (0, 0)),
        ],
        out_specs=pl.BlockSpec((BR, Vp), lambda i: (i, 0)),
        out_shape=jax.ShapeDtypeStruct((B, Vp), jnp.float32),
        compiler_params=pltpu.CompilerParams(
            dimension_semantics=("parallel",)),
    )(xp, params)

    return logits, mask[:, :V]


# speedup vs baseline: 1.2096x; 1.2096x over previous
import functools

import jax
import jax.numpy as jnp
from jax.experimental import pallas as pl
from jax.experimental.pallas import tpu as pltpu

_VALUE_ITERS = 32
_SIZE_ITERS = 18
_INDEX_ITERS = 18


def _conformal_body(x_ref, p_ref, o_ref):
    invT = p_ref[0, 0]
    qhat = p_ref[0, 1]
    lam = p_ref[0, 2]
    kreg = p_ref[0, 3]

    x = x_ref[...] * invT
    m = jnp.max(x, axis=1, keepdims=True)
    e = jnp.exp(x - m)
    z = jnp.sum(e, axis=1, keepdims=True)
    zq = qhat * z
    zlam = lam * z

    lo = jnp.zeros_like(z)
    hi = jnp.full_like(z, 1.5)

    def vstep(_, carry):
        lo, hi = carry
        mid = 0.5 * (lo + hi)
        pred = e >= mid
        n = jnp.sum(jnp.where(pred, 1.0, 0.0), axis=1, keepdims=True)
        s = jnp.sum(jnp.where(pred, e, 0.0), axis=1, keepdims=True)
        g = s + zlam * jnp.maximum(n - kreg, 0.0)
        big = g > zq
        lo = jnp.where(big, mid, lo)
        hi = jnp.where(big, hi, mid)
        return lo, hi

    lo, hi = jax.lax.fori_loop(0, _VALUE_ITERS, vstep, (lo, hi))

    inc = e >= lo
    v = jnp.min(jnp.where(inc, e, 2.0), axis=1, keepdims=True)
    strict = e > v
    n_strict = jnp.sum(jnp.where(strict, 1.0, 0.0), axis=1, keepdims=True)
    s_strict = jnp.sum(jnp.where(strict, e, 0.0), axis=1, keepdims=True)
    n_total = jnp.sum(jnp.where(e >= v, 1.0, 0.0), axis=1, keepdims=True)

    def nstep(_, carry):
        lo_n, hi_n = carry
        mid = jnp.floor(0.5 * (lo_n + hi_n))
        g = s_strict + (mid - n_strict) * v + zlam * jnp.maximum(mid - kreg, 0.0)
        big = (g > zq) & (mid > lo_n)
        hi_n = jnp.where(big, mid, hi_n)
        lo_n = jnp.where(big | (mid <= lo_n), lo_n, mid)
        return lo_n, hi_n

    _, n_star = jax.lax.fori_loop(0, _SIZE_ITERS, nstep, (n_strict, n_total))
    needed = n_star - n_strict

    iota = jax.lax.broadcasted_iota(jnp.int32, e.shape, 1).astype(jnp.float32)
    tie = e == v
    full_cut = jnp.full_like(z, float(e.shape[1]))

    def _bisect_cutoff():
        def istep(_, carry):
            lo_i, hi_i = carry
            mid = jnp.floor(0.5 * (lo_i + hi_i))
            cnt = jnp.sum(jnp.where(tie & (iota < mid), 1.0, 0.0), axis=1,
                          keepdims=True)
            ok = (cnt >= needed) & (mid > lo_i)
            hi_i = jnp.where(ok, mid, hi_i)
            lo_i = jnp.where(ok | (mid <= lo_i), lo_i, mid)
            return lo_i, hi_i

        return jax.lax.fori_loop(0, _INDEX_ITERS, istep,
                                 (jnp.zeros_like(z), full_cut))[1]

    has_partial = jnp.any(needed < n_total - n_strict)
    cutoff = jax.lax.cond(has_partial, _bisect_cutoff, lambda: full_cut)

    mask = (strict | (tie & (iota < cutoff))).astype(jnp.float32)
    mask = jnp.where(qhat == 1.0, 1.0, mask)
    o_ref[...] = mask


@functools.partial(jax.jit, static_argnames=())
def kernel(logits, temperature, penalties, qhat):
    B, V = logits.shape
    Vp = ((V + 127) // 128) * 128
    BR = 16

    lam = penalties[0, -1]
    kreg = jnp.sum(penalties[0] == 0).astype(jnp.float32)
    invT = 1.0 / temperature[0]
    params = jnp.stack(
        [invT, qhat.astype(jnp.float32), lam, kreg]
    ).reshape(1, 4)

    xp = jnp.pad(logits, ((0, 0), (0, Vp - V)), constant_values=-jnp.inf)

    mask = pl.pallas_call(
        _conformal_body,
        grid=(B // BR,),
        in_specs=[
            pl.BlockSpec((BR, Vp), lambda i: (i, 0)),
            pl.BlockSpec((1, 4), lambda i: (0, 0)),
        ],
        out_specs=pl.BlockSpec((BR, Vp), lambda i: (i, 0)),
        out_shape=jax.ShapeDtypeStruct((B, Vp), jnp.float32),
        compiler_params=pltpu.CompilerParams(
            dimension_semantics=("parallel",)),
    )(xp, params)

    return logits, mask[:, :V]
